# baseline (speedup 1.0000x reference)
import functools

import jax
import jax.numpy as jnp
from jax import lax
from jax.experimental import pallas as pl
from jax.experimental.pallas import tpu as pltpu
from jax.experimental.pallas import tpu_sc as plsc

_N = 10000
_E = 320000
_F_IN = 128
_HID = 128
_F_OUT = 64

_NC = 2
_NS = 16
_L = 16
_NW = _NC * _NS
_EPW = _E // _NW
_ECH = 80
_NCH = _EPW // _ECH
_NPT = 624
_NREM = _N - _NPT * _NS
_DPAD = 16

_BN = 400
_GRID = _N // _BN

_DNUMS = lax.GatherDimensionNumbers(
    offset_dims=(), collapsed_slice_dims=(0,), start_index_map=(0,))
_IN_BOUNDS = lax.GatherScatterMode.PROMISE_IN_BOUNDS


def _bcast_lane(vec, lane):
    return lax.gather(vec, jnp.full((_L, 1), lane, jnp.int32), _DNUMS, (1,),
                      mode=_IN_BOUNDS)


def _zero_rows(ref, nrows, nvr):
    def body(r, _):
        for j in range(nvr):
            ref[r, pl.ds(j * _L, _L)] = jnp.zeros((_L,), jnp.float32)
        return 0
    lax.fori_loop(0, nrows, body, 0)


def _zero_acc_hbm(z0, acc, s):
    rbase = s * _NPT
    pltpu.sync_copy(z0.at[pl.ds(rbase, _NPT)], acc.at[pl.ds(rbase, _NPT)])

    @pl.when(s == _NS - 1)
    def _():
        pltpu.sync_copy(z0.at[pl.ds(_NPT * _NS, _NREM)],
                        acc.at[pl.ds(_NPT * _NS, _NREM)])


def _zero_acc_slice(zsrc, acc, s):
    rbase = s * _NPT
    for k in range(_NPT // _ECH):
        pltpu.sync_copy(zsrc, acc.at[pl.ds(rbase + k * _ECH, _ECH)])
    rem = _NPT % _ECH
    pltpu.sync_copy(zsrc.at[pl.ds(0, rem)],
                    acc.at[pl.ds(rbase + (_NPT // _ECH) * _ECH, rem)])

    @pl.when(s == _NS - 1)
    def _():
        pltpu.sync_copy(zsrc.at[pl.ds(0, _NREM)],
                        acc.at[pl.ds(_NPT * _NS, _NREM)])


def _dump_acc_slice(acc, out_hbm, c, s):
    rbase = s * _NPT
    pltpu.sync_copy(acc.at[pl.ds(rbase, _NPT)],
                    out_hbm.at[c, pl.ds(rbase, _NPT)])

    @pl.when(s == _NS - 1)
    def _():
        pltpu.sync_copy(acc.at[pl.ds(_NPT * _NS, _NREM)],
                        out_hbm.at[c, pl.ds(_NPT * _NS, _NREM)])


def _make_deg():
    mesh = plsc.VectorSubcoreMesh(core_axis_name="c", subcore_axis_name="s")

    @functools.partial(
        pl.kernel,
        mesh=mesh,
        compiler_params=pltpu.CompilerParams(use_tc_tiling_on_sc=False),
        out_type=jax.ShapeDtypeStruct((_NC, _N, _DPAD), jnp.float32),
        scratch_types=[
            pltpu.VMEM((_NCH, 2, _ECH), jnp.int32),
            pltpu.VMEM((_EPW,), jnp.float32),
            pltpu.VMEM((_ECH, _DPAD), jnp.float32),
            pltpu.VMEM((_ECH, _DPAD), jnp.float32),
            pltpu.VMEM_SHARED((_N, _DPAD), jnp.float32),
            pltpu.SemaphoreType.DMA,
            pltpu.SemaphoreType.DMA,
        ],
    )
    def deg(ed_hbm, ew_hbm, z0_hbm, out_hbm, ed_all, ew_all, stage0, stage1,
            dacc, ssem0, ssem1):
        c = lax.axis_index("c")
        s = lax.axis_index("s")
        wid = s * _NC + c
        stages = (stage0, stage1)
        ssems = (ssem0, ssem1)

        pltpu.sync_copy(ed_hbm.at[pl.ds(wid * _NCH, _NCH)], ed_all)
        pltpu.sync_copy(ew_hbm.at[pl.ds(wid * _EPW, _EPW)], ew_all)
        _zero_acc_hbm(z0_hbm, dacc, s)
        plsc.subcore_barrier()

        lane_idx = lax.iota(jnp.int32, _L)
        zerov = jnp.zeros((_L,), jnp.float32)

        def build(ci, buf):
            def grp(g, _):
                ewv = ew_all[pl.ds(ci * _ECH + g * _L, _L)]
                for lane in range(_L):
                    bc = _bcast_lane(ewv, lane)
                    buf[g * _L + lane, pl.ds(0, _L)] = jnp.where(
                        lane_idx == 0, bc, zerov)
                return 0
            lax.fori_loop(0, _ECH // _L, grp, 0)

        def scatter_of(ci, b):
            return pltpu.make_async_copy(
                stages[b], dacc.at[ed_all.at[ci, 1]], ssems[b])

        @pl.loop(0, _NCH - 1, step=2)
        def _(g):
            for b in range(2):
                ci = g + b

                @pl.when(ci >= 2)
                def _():
                    scatter_of(ci - 2, b).wait()

                build(ci, stages[b])
                pltpu.async_copy(stages[b], dacc.at[ed_all.at[ci, 1]],
                                 ssems[b], add=True)

        ci_t = _NCH - 1
        scatter_of(ci_t - 2, 0).wait()
        build(ci_t, stages[0])
        pltpu.async_copy(stages[0], dacc.at[ed_all.at[ci_t, 1]], ssems[0],
                         add=True)

        scatter_of(ci_t - 1, 1).wait()
        scatter_of(ci_t, 0).wait()
        plsc.subcore_barrier()
        _dump_acc_slice(dacc, out_hbm, c, s)

    return deg


def _make_agg(D, nact=None):
    mesh = plsc.VectorSubcoreMesh(core_axis_name="c", subcore_axis_name="s")
    nvr = D // _L
    nact = nvr if nact is None else nact

    @functools.partial(
        pl.kernel,
        mesh=mesh,
        compiler_params=pltpu.CompilerParams(use_tc_tiling_on_sc=False),
        out_type=jax.ShapeDtypeStruct((_NC, _N, D), jnp.float32),
        scratch_types=[
            pltpu.VMEM((_NCH, 2, _ECH), jnp.int32),
            pltpu.VMEM((_EPW,), jnp.float32),
            pltpu.VMEM((_ECH, D), jnp.float32),
            pltpu.VMEM((_ECH, D), jnp.float32),
            pltpu.VMEM_SHARED((_N, D), jnp.float32),
            pltpu.SemaphoreType.DMA,
            pltpu.SemaphoreType.DMA,
            pltpu.SemaphoreType.DMA,
            pltpu.SemaphoreType.DMA,
            pltpu.SemaphoreType.DMA,
            pltpu.SemaphoreType.DMA,
        ],
    )
    def agg(z_hbm, ed_hbm, ew_hbm, z0_hbm, out_hbm, ed_all, ew_all, rows0,
            rows1, acc, gsem0, gsem1, gsem0b, gsem1b, ssem0, ssem1):
        c = lax.axis_index("c")
        s = lax.axis_index("s")
        wid = s * _NC + c
        rows = (rows0, rows1)
        gsems = (gsem0, gsem1)
        gsemsb = (gsem0b, gsem1b)
        ssems = (ssem0, ssem1)

        pltpu.sync_copy(ed_hbm.at[pl.ds(wid * _NCH, _NCH)], ed_all)
        pltpu.sync_copy(ew_hbm.at[pl.ds(wid * _EPW, _EPW)], ew_all)
        _zero_acc_hbm(z0_hbm, acc, s)

        _EH2 = _ECH // 2

        def gather_of(ci, b):
            return (
                pltpu.make_async_copy(
                    z_hbm.at[ed_all.at[ci, 0, pl.ds(0, _EH2)]],
                    rows[b].at[pl.ds(0, _EH2)], gsems[b]),
                pltpu.make_async_copy(
                    z_hbm.at[ed_all.at[ci, 0, pl.ds(_EH2, _EH2)]],
                    rows[b].at[pl.ds(_EH2, _EH2)], gsemsb[b]),
            )

        def gather_start(ci, b):
            for d in gather_of(ci, b):
                d.start()

        def gather_wait(ci, b):
            for d in gather_of(ci, b):
                d.wait()

        def scatter_of(ci, b):
            return pltpu.make_async_copy(rows[b], acc.at[ed_all.at[ci, 1]],
                                         ssems[b])

        gather_start(0, 0)
        plsc.subcore_barrier()

        def scale(ci, buf):
            def grp(g, _):
                ewv = ew_all[pl.ds(ci * _ECH + g * _L, _L)]
                for lane in range(_L):
                    bc = _bcast_lane(ewv, lane)
                    e = g * _L + lane
                    for j in range(nact):
                        buf[e, pl.ds(j * _L, _L)] = (
                            buf[e, pl.ds(j * _L, _L)] * bc)
                return 0
            lax.fori_loop(0, _ECH // _L, grp, 0)

        @pl.loop(0, _NCH - 1, step=2)
        def _(g):
            for b in range(2):
                ci = g + b
                nb = 1 - b

                @pl.when(ci >= 1)
                def _():
                    scatter_of(ci - 1, nb).wait()

                gather_start(ci + 1, nb)
                gather_wait(ci, b)
                scale(ci, rows[b])
                pltpu.async_copy(rows[b], acc.at[ed_all.at[ci, 1]], ssems[b],
                                 add=True)

        ci_t = _NCH - 1
        gather_wait(ci_t, 0)
        scale(ci_t, rows[0])
        pltpu.async_copy(rows[0], acc.at[ed_all.at[ci_t, 1]], ssems[0],
                         add=True)

        scatter_of(ci_t - 1, 1).wait()
        scatter_of(ci_t, 0).wait()
        plsc.subcore_barrier()
        _dump_acc_slice(acc, out_hbm, c, s)

    return agg


def _dinv_of(degp):
    deg = 1.0 + degp[0, :, 0] + degp[1, :, 0]
    return lax.rsqrt(deg)


def _z1_body(degp_ref, x_ref, w_ref, o_ref):
    dinv = _dinv_of(degp_ref[...])
    xw = jnp.dot(x_ref[...], w_ref[...], preferred_element_type=jnp.float32)
    o_ref[...] = dinv[:, None] * xw


def _z2_body(degp_ref, aggp_ref, z1_ref, b1_ref, w2_ref, o_ref):
    dinv = _dinv_of(degp_ref[...])
    a = aggp_ref[...]
    h = dinv[:, None] * (a[0] + a[1] + z1_ref[...]) + b1_ref[...][None, :]
    h = jnp.maximum(h, 0.0)
    hw = jnp.dot(h, w2_ref[...], preferred_element_type=jnp.float32)
    o_ref[...] = jnp.concatenate(
        [dinv[:, None] * hw, jnp.zeros((_BN, _HID - _F_OUT), jnp.float32)],
        axis=1)


def _out_body(degp_ref, aggp_ref, z2_ref, b2_ref, o_ref):
    dinv = _dinv_of(degp_ref[...])
    a = aggp_ref[...]
    t = (a[0] + a[1] + z2_ref[...])[:, :_F_OUT]
    o_ref[...] = dinv[:, None] * t + b2_ref[...][None, :]


_degp_spec = pl.BlockSpec((2, _BN, _DPAD), lambda i: (0, i, 0))


def _z1(degp, x, W1):
    return pl.pallas_call(
        _z1_body,
        grid=(_GRID,),
        in_specs=[
            _degp_spec,
            pl.BlockSpec((_BN, _F_IN), lambda i: (i, 0)),
            pl.BlockSpec((_F_IN, _HID), lambda i: (0, 0)),
        ],
        out_specs=pl.BlockSpec((_BN, _HID), lambda i: (i, 0)),
        out_shape=jax.ShapeDtypeStruct((_N, _HID), jnp.float32),
    )(degp, x, W1)


def _z2(degp, aggp, z1, b1, W2):
    return pl.pallas_call(
        _z2_body,
        grid=(_GRID,),
        in_specs=[
            _degp_spec,
            pl.BlockSpec((2, _BN, _HID), lambda i: (0, i, 0)),
            pl.BlockSpec((_BN, _HID), lambda i: (i, 0)),
            pl.BlockSpec((_HID,), lambda i: (0,)),
            pl.BlockSpec((_HID, _F_OUT), lambda i: (0, 0)),
        ],
        out_specs=pl.BlockSpec((_BN, _HID), lambda i: (i, 0)),
        out_shape=jax.ShapeDtypeStruct((_N, _HID), jnp.float32),
    )(degp, aggp, z1, b1, W2)


def _out(degp, aggp, z2, b2):
    return pl.pallas_call(
        _out_body,
        grid=(_GRID,),
        in_specs=[
            _degp_spec,
            pl.BlockSpec((2, _BN, _HID), lambda i: (0, i, 0)),
            pl.BlockSpec((_BN, _HID), lambda i: (i, 0)),
            pl.BlockSpec((_F_OUT,), lambda i: (0,)),
        ],
        out_specs=pl.BlockSpec((_BN, _F_OUT), lambda i: (i, 0)),
        out_shape=jax.ShapeDtypeStruct((_N, _F_OUT), jnp.float32),
    )(degp, aggp, z2, b2)


_deg_kernel = _make_deg()
_agg128 = _make_agg(_HID)
_agg64 = _make_agg(_HID, nact=_F_OUT // _L)


def kernel(x, edge_index, edge_weight, W1, b1, W2, b2):
    ed = jnp.stack([edge_index[0], edge_index[1]], axis=0)
    ed = ed.reshape(2, _NW * _NCH, _ECH).transpose(1, 0, 2)

    z0_128 = jnp.zeros((_N, _HID), jnp.float32)
    z0_16 = jnp.zeros((_N, _DPAD), jnp.float32)

    degp = _deg_kernel(ed, edge_weight, z0_16)
    z1 = _z1(degp, x, W1)
    aggp1 = _agg128(z1, ed, edge_weight, z0_128)
    z2 = _z2(degp, aggp1, z1, b1, W2)
    aggp2 = _agg64(z2, ed, edge_weight, z0_128)
    return _out(degp, aggp2, z2, b2)


# speedup vs baseline: 1.0008x; 1.0008x over previous
import functools

import jax
import jax.numpy as jnp
from jax import lax
from jax.experimental import pallas as pl
from jax.experimental.pallas import tpu as pltpu
from jax.experimental.pallas import tpu_sc as plsc

_N = 10000
_E = 320000
_F_IN = 128
_HID = 128
_F_OUT = 64

_NC = 2
_NS = 16
_L = 16
_NW = _NC * _NS
_EPW = _E // _NW
_ECH = 80
_NCH = _EPW // _ECH
_NPT = 624
_NREM = _N - _NPT * _NS
_DPAD = 16

_BN = 400
_GRID = _N // _BN

_DNUMS = lax.GatherDimensionNumbers(
    offset_dims=(), collapsed_slice_dims=(0,), start_index_map=(0,))
_IN_BOUNDS = lax.GatherScatterMode.PROMISE_IN_BOUNDS


def _bcast_lane(vec, lane):
    return lax.gather(vec, jnp.full((_L, 1), lane, jnp.int32), _DNUMS, (1,),
                      mode=_IN_BOUNDS)


def _zero_rows(ref, nrows, nvr):
    def body(r, _):
        for j in range(nvr):
            ref[r, pl.ds(j * _L, _L)] = jnp.zeros((_L,), jnp.float32)
        return 0
    lax.fori_loop(0, nrows, body, 0)


def _zero_acc_hbm(z0, acc, s):
    rbase = s * _NPT
    pltpu.sync_copy(z0.at[pl.ds(rbase, _NPT)], acc.at[pl.ds(rbase, _NPT)])

    @pl.when(s == _NS - 1)
    def _():
        pltpu.sync_copy(z0.at[pl.ds(_NPT * _NS, _NREM)],
                        acc.at[pl.ds(_NPT * _NS, _NREM)])


def _zero_acc_slice(zsrc, acc, s):
    rbase = s * _NPT
    for k in range(_NPT // _ECH):
        pltpu.sync_copy(zsrc, acc.at[pl.ds(rbase + k * _ECH, _ECH)])
    rem = _NPT % _ECH
    pltpu.sync_copy(zsrc.at[pl.ds(0, rem)],
                    acc.at[pl.ds(rbase + (_NPT // _ECH) * _ECH, rem)])

    @pl.when(s == _NS - 1)
    def _():
        pltpu.sync_copy(zsrc.at[pl.ds(0, _NREM)],
                        acc.at[pl.ds(_NPT * _NS, _NREM)])


def _dump_acc_slice(acc, out_hbm, c, s):
    rbase = s * _NPT
    pltpu.sync_copy(acc.at[pl.ds(rbase, _NPT)],
                    out_hbm.at[c, pl.ds(rbase, _NPT)])

    @pl.when(s == _NS - 1)
    def _():
        pltpu.sync_copy(acc.at[pl.ds(_NPT * _NS, _NREM)],
                        out_hbm.at[c, pl.ds(_NPT * _NS, _NREM)])


def _make_deg():
    mesh = plsc.VectorSubcoreMesh(core_axis_name="c", subcore_axis_name="s")

    @functools.partial(
        pl.kernel,
        mesh=mesh,
        compiler_params=pltpu.CompilerParams(use_tc_tiling_on_sc=False),
        out_type=jax.ShapeDtypeStruct((_NC, _N, _DPAD), jnp.float32),
        scratch_types=[
            pltpu.VMEM((_NCH, 2, _ECH), jnp.int32),
            pltpu.VMEM((_EPW,), jnp.float32),
            pltpu.VMEM((_ECH, _DPAD), jnp.float32),
            pltpu.VMEM((_ECH, _DPAD), jnp.float32),
            pltpu.VMEM_SHARED((_N, _DPAD), jnp.float32),
            pltpu.SemaphoreType.DMA,
            pltpu.SemaphoreType.DMA,
        ],
    )
    def deg(ed_hbm, ew_hbm, z0_hbm, out_hbm, ed_all, ew_all, stage0, stage1,
            dacc, ssem0, ssem1):
        c = lax.axis_index("c")
        s = lax.axis_index("s")
        wid = s * _NC + c
        stages = (stage0, stage1)
        ssems = (ssem0, ssem1)

        pltpu.sync_copy(ed_hbm.at[pl.ds(wid * _NCH, _NCH)], ed_all)
        pltpu.sync_copy(ew_hbm.at[pl.ds(wid * _EPW, _EPW)], ew_all)
        _zero_acc_hbm(z0_hbm, dacc, s)
        plsc.subcore_barrier()

        lane_idx = lax.iota(jnp.int32, _L)
        zerov = jnp.zeros((_L,), jnp.float32)

        def build(ci, buf):
            def grp(g, _):
                ewv = ew_all[pl.ds(ci * _ECH + g * _L, _L)]
                for lane in range(_L):
                    bc = _bcast_lane(ewv, lane)
                    buf[g * _L + lane, pl.ds(0, _L)] = jnp.where(
                        lane_idx == 0, bc, zerov)
                return 0
            lax.fori_loop(0, _ECH // _L, grp, 0)

        def scatter_of(ci, b):
            return pltpu.make_async_copy(
                stages[b], dacc.at[ed_all.at[ci, 1]], ssems[b])

        @pl.loop(0, _NCH - 1, step=2)
        def _(g):
            for b in range(2):
                ci = g + b

                @pl.when(ci >= 2)
                def _():
                    scatter_of(ci - 2, b).wait()

                build(ci, stages[b])
                pltpu.async_copy(stages[b], dacc.at[ed_all.at[ci, 1]],
                                 ssems[b], add=True)

        ci_t = _NCH - 1
        scatter_of(ci_t - 2, 0).wait()
        build(ci_t, stages[0])
        pltpu.async_copy(stages[0], dacc.at[ed_all.at[ci_t, 1]], ssems[0],
                         add=True)

        scatter_of(ci_t - 1, 1).wait()
        scatter_of(ci_t, 0).wait()
        plsc.subcore_barrier()
        _dump_acc_slice(dacc, out_hbm, c, s)

    return deg


def _make_agg(D, nact=None):
    mesh = plsc.VectorSubcoreMesh(core_axis_name="c", subcore_axis_name="s")
    nvr = D // _L
    nact = nvr if nact is None else nact

    @functools.partial(
        pl.kernel,
        mesh=mesh,
        compiler_params=pltpu.CompilerParams(use_tc_tiling_on_sc=False),
        out_type=jax.ShapeDtypeStruct((_NC, _N, D), jnp.float32),
        scratch_types=[
            pltpu.VMEM((_NCH, 2, _ECH), jnp.int32),
            pltpu.VMEM((_EPW,), jnp.float32),
            pltpu.VMEM((_ECH, D), jnp.float32),
            pltpu.VMEM((_ECH, D), jnp.float32),
            pltpu.VMEM_SHARED((_N, D), jnp.float32),
            pltpu.SemaphoreType.DMA,
            pltpu.SemaphoreType.DMA,
            pltpu.SemaphoreType.DMA,
            pltpu.SemaphoreType.DMA,
            pltpu.SemaphoreType.DMA,
            pltpu.SemaphoreType.DMA,
        ],
    )
    def agg(z_hbm, ed_hbm, ew_hbm, z0_hbm, out_hbm, ed_all, ew_all, rows0,
            rows1, acc, gsem0, gsem1, gsem0b, gsem1b, ssem0, ssem1):
        c = lax.axis_index("c")
        s = lax.axis_index("s")
        wid = s * _NC + c
        rows = (rows0, rows1)
        gsems = (gsem0, gsem1)
        gsemsb = (gsem0b, gsem1b)
        ssems = (ssem0, ssem1)

        pltpu.sync_copy(ed_hbm.at[pl.ds(wid * _NCH, _NCH)], ed_all)
        pltpu.sync_copy(ew_hbm.at[pl.ds(wid * _EPW, _EPW)], ew_all)
        _zero_acc_hbm(z0_hbm, acc, s)

        _EH2 = _ECH // 2

        def gather_of(ci, b):
            return (
                pltpu.make_async_copy(
                    z_hbm.at[ed_all.at[ci, 0, pl.ds(0, _EH2)]],
                    rows[b].at[pl.ds(0, _EH2)], gsems[b]),
                pltpu.make_async_copy(
                    z_hbm.at[ed_all.at[ci, 0, pl.ds(_EH2, _EH2)]],
                    rows[b].at[pl.ds(_EH2, _EH2)], gsemsb[b]),
            )

        def gather_start(ci, b):
            for d in gather_of(ci, b):
                d.start()

        def gather_wait(ci, b):
            for d in gather_of(ci, b):
                d.wait()

        def scatter_of(ci, b):
            return pltpu.make_async_copy(rows[b], acc.at[ed_all.at[ci, 1]],
                                         ssems[b])

        gather_start(0, 0)
        plsc.subcore_barrier()

        def scale(ci, buf):
            def grp(g, _):
                ewv = ew_all[pl.ds(ci * _ECH + g * _L, _L)]
                for lane in range(_L):
                    bc = _bcast_lane(ewv, lane)
                    e = g * _L + lane
                    for j in range(nact):
                        buf[e, pl.ds(j * _L, _L)] = (
                            buf[e, pl.ds(j * _L, _L)] * bc)
                return 0
            lax.fori_loop(0, _ECH // _L, grp, 0)

        @pl.loop(0, _NCH - 1, step=2)
        def _(g):
            for b in range(2):
                ci = g + b
                nb = 1 - b

                @pl.when(ci >= 1)
                def _():
                    scatter_of(ci - 1, nb).wait()

                gather_start(ci + 1, nb)
                gather_wait(ci, b)
                scale(ci, rows[b])
                pltpu.async_copy(rows[b], acc.at[ed_all.at[ci, 1]], ssems[b],
                                 add=True)

        ci_t = _NCH - 1
        gather_wait(ci_t, 0)
        scale(ci_t, rows[0])
        pltpu.async_copy(rows[0], acc.at[ed_all.at[ci_t, 1]], ssems[0],
                         add=True)

        scatter_of(ci_t - 1, 1).wait()
        scatter_of(ci_t, 0).wait()
        plsc.subcore_barrier()
        _dump_acc_slice(acc, out_hbm, c, s)

    return agg


def _dinv_of(degp):
    deg = 1.0 + degp[0, :, 0] + degp[1, :, 0]
    return lax.rsqrt(deg)


def _mm_body(x_ref, w_ref, o_ref):
    o_ref[...] = jnp.dot(x_ref[...], w_ref[...],
                         preferred_element_type=jnp.float32)


def _scale_body(degp_ref, xw_ref, o_ref):
    dinv = _dinv_of(degp_ref[...])
    o_ref[...] = dinv[:, None] * xw_ref[...]


def _z2_body(degp_ref, aggp_ref, z1_ref, b1_ref, w2_ref, o_ref):
    dinv = _dinv_of(degp_ref[...])
    a = aggp_ref[...]
    h = dinv[:, None] * (a[0] + a[1] + z1_ref[...]) + b1_ref[...][None, :]
    h = jnp.maximum(h, 0.0)
    hw = jnp.dot(h, w2_ref[...], preferred_element_type=jnp.float32)
    o_ref[...] = jnp.concatenate(
        [dinv[:, None] * hw, jnp.zeros((_BN, _HID - _F_OUT), jnp.float32)],
        axis=1)


def _out_body(degp_ref, aggp_ref, z2_ref, b2_ref, o_ref):
    dinv = _dinv_of(degp_ref[...])
    a = aggp_ref[...]
    t = (a[0] + a[1] + z2_ref[...])[:, :_F_OUT]
    o_ref[...] = dinv[:, None] * t + b2_ref[...][None, :]


_degp_spec = pl.BlockSpec((2, _BN, _DPAD), lambda i: (0, i, 0))


def _mm(x, W):
    return pl.pallas_call(
        _mm_body,
        grid=(_GRID,),
        in_specs=[
            pl.BlockSpec((_BN, _F_IN), lambda i: (i, 0)),
            pl.BlockSpec((_F_IN, _HID), lambda i: (0, 0)),
        ],
        out_specs=pl.BlockSpec((_BN, _HID), lambda i: (i, 0)),
        out_shape=jax.ShapeDtypeStruct((_N, _HID), jnp.float32),
    )(x, W)


def _z1(degp, xw):
    return pl.pallas_call(
        _scale_body,
        grid=(_GRID,),
        in_specs=[
            _degp_spec,
            pl.BlockSpec((_BN, _HID), lambda i: (i, 0)),
        ],
        out_specs=pl.BlockSpec((_BN, _HID), lambda i: (i, 0)),
        out_shape=jax.ShapeDtypeStruct((_N, _HID), jnp.float32),
    )(degp, xw)


def _z2(degp, aggp, z1, b1, W2):
    return pl.pallas_call(
        _z2_body,
        grid=(_GRID,),
        in_specs=[
            _degp_spec,
            pl.BlockSpec((2, _BN, _HID), lambda i: (0, i, 0)),
            pl.BlockSpec((_BN, _HID), lambda i: (i, 0)),
            pl.BlockSpec((_HID,), lambda i: (0,)),
            pl.BlockSpec((_HID, _F_OUT), lambda i: (0, 0)),
        ],
        out_specs=pl.BlockSpec((_BN, _HID), lambda i: (i, 0)),
        out_shape=jax.ShapeDtypeStruct((_N, _HID), jnp.float32),
    )(degp, aggp, z1, b1, W2)


def _out(degp, aggp, z2, b2):
    return pl.pallas_call(
        _out_body,
        grid=(_GRID,),
        in_specs=[
            _degp_spec,
            pl.BlockSpec((2, _BN, _HID), lambda i: (0, i, 0)),
            pl.BlockSpec((_BN, _HID), lambda i: (i, 0)),
            pl.BlockSpec((_F_OUT,), lambda i: (0,)),
        ],
        out_specs=pl.BlockSpec((_BN, _F_OUT), lambda i: (i, 0)),
        out_shape=jax.ShapeDtypeStruct((_N, _F_OUT), jnp.float32),
    )(degp, aggp, z2, b2)


_deg_kernel = _make_deg()
_agg128 = _make_agg(_HID)
_agg64 = _make_agg(_HID, nact=_F_OUT // _L)


def kernel(x, edge_index, edge_weight, W1, b1, W2, b2):
    ed = jnp.stack([edge_index[0], edge_index[1]], axis=0)
    ed = ed.reshape(2, _NW * _NCH, _ECH).transpose(1, 0, 2)

    z0_128 = jnp.zeros((_N, _HID), jnp.float32)
    z0_16 = jnp.zeros((_N, _DPAD), jnp.float32)

    degp = _deg_kernel(ed, edge_weight, z0_16)
    xw1 = _mm(x, W1)
    z1 = _z1(degp, xw1)
    aggp1 = _agg128(z1, ed, edge_weight, z0_128)
    z2 = _z2(degp, aggp1, z1, b1, W2)
    aggp2 = _agg64(z2, ed, edge_weight, z0_128)
    return _out(degp, aggp2, z2, b2)


# speedup vs baseline: 1.0206x; 1.0198x over previous
import functools

import jax
import jax.numpy as jnp
from jax import lax
from jax.experimental import pallas as pl
from jax.experimental.pallas import tpu as pltpu
from jax.experimental.pallas import tpu_sc as plsc

_N = 10000
_E = 320000
_F_IN = 128
_HID = 128
_F_OUT = 64

_NC = 2
_NS = 16
_L = 16
_NW = _NC * _NS
_EPW = _E // _NW
_ECH = 80
_NCH = _EPW // _ECH
_NPT = 624
_NREM = _N - _NPT * _NS
_DPAD = 16

_BN = 400
_GRID = _N // _BN

_DNUMS = lax.GatherDimensionNumbers(
    offset_dims=(), collapsed_slice_dims=(0,), start_index_map=(0,))
_IN_BOUNDS = lax.GatherScatterMode.PROMISE_IN_BOUNDS


def _bcast_lane(vec, lane):
    return lax.gather(vec, jnp.full((_L, 1), lane, jnp.int32), _DNUMS, (1,),
                      mode=_IN_BOUNDS)


def _zero_rows(ref, nrows, nvr):
    def body(r, _):
        for j in range(nvr):
            ref[r, pl.ds(j * _L, _L)] = jnp.zeros((_L,), jnp.float32)
        return 0
    lax.fori_loop(0, nrows, body, 0)


def _zero_acc_hbm(z0, acc, s):
    rbase = s * _NPT
    pltpu.sync_copy(z0.at[pl.ds(rbase, _NPT)], acc.at[pl.ds(rbase, _NPT)])

    @pl.when(s == _NS - 1)
    def _():
        pltpu.sync_copy(z0.at[pl.ds(_NPT * _NS, _NREM)],
                        acc.at[pl.ds(_NPT * _NS, _NREM)])


def _zero_acc_slice(zsrc, acc, s):
    rbase = s * _NPT
    for k in range(_NPT // _ECH):
        pltpu.sync_copy(zsrc, acc.at[pl.ds(rbase + k * _ECH, _ECH)])
    rem = _NPT % _ECH
    pltpu.sync_copy(zsrc.at[pl.ds(0, rem)],
                    acc.at[pl.ds(rbase + (_NPT // _ECH) * _ECH, rem)])

    @pl.when(s == _NS - 1)
    def _():
        pltpu.sync_copy(zsrc.at[pl.ds(0, _NREM)],
                        acc.at[pl.ds(_NPT * _NS, _NREM)])


def _dump_acc_slice(acc, out_hbm, c, s):
    rbase = s * _NPT
    pltpu.sync_copy(acc.at[pl.ds(rbase, _NPT)],
                    out_hbm.at[c, pl.ds(rbase, _NPT)])

    @pl.when(s == _NS - 1)
    def _():
        pltpu.sync_copy(acc.at[pl.ds(_NPT * _NS, _NREM)],
                        out_hbm.at[c, pl.ds(_NPT * _NS, _NREM)])


def _make_deg():
    mesh = plsc.VectorSubcoreMesh(core_axis_name="c", subcore_axis_name="s")

    @functools.partial(
        pl.kernel,
        mesh=mesh,
        compiler_params=pltpu.CompilerParams(use_tc_tiling_on_sc=False),
        out_type=jax.ShapeDtypeStruct((_NC, _N, _DPAD), jnp.float32),
        scratch_types=[
            pltpu.VMEM((_NCH, 2, _ECH), jnp.int32),
            pltpu.VMEM((_EPW,), jnp.float32),
            pltpu.VMEM((_ECH, _DPAD), jnp.float32),
            pltpu.VMEM((_ECH, _DPAD), jnp.float32),
            pltpu.VMEM_SHARED((_N, _DPAD), jnp.float32),
            pltpu.SemaphoreType.DMA,
            pltpu.SemaphoreType.DMA,
        ],
    )
    def deg(ed_hbm, ew_hbm, out_hbm, ed_all, ew_all, stage0, stage1,
            dacc, ssem0, ssem1):
        c = lax.axis_index("c")
        s = lax.axis_index("s")
        wid = s * _NC + c
        stages = (stage0, stage1)
        ssems = (ssem0, ssem1)

        pltpu.sync_copy(ed_hbm.at[pl.ds(wid * _NCH, _NCH)], ed_all)
        pltpu.sync_copy(ew_hbm.at[pl.ds(wid * _EPW, _EPW)], ew_all)
        _zero_rows(stage0, _ECH, _DPAD // _L)
        _zero_acc_slice(stage0, dacc, s)
        plsc.subcore_barrier()

        lane_idx = lax.iota(jnp.int32, _L)
        zerov = jnp.zeros((_L,), jnp.float32)

        def build(ci, buf):
            def grp(g, _):
                ewv = ew_all[pl.ds(ci * _ECH + g * _L, _L)]
                for lane in range(_L):
                    bc = _bcast_lane(ewv, lane)
                    buf[g * _L + lane, pl.ds(0, _L)] = jnp.where(
                        lane_idx == 0, bc, zerov)
                return 0
            lax.fori_loop(0, _ECH // _L, grp, 0)

        def scatter_of(ci, b):
            return pltpu.make_async_copy(
                stages[b], dacc.at[ed_all.at[ci, 1]], ssems[b])

        @pl.loop(0, _NCH - 1, step=2)
        def _(g):
            for b in range(2):
                ci = g + b

                @pl.when(ci >= 2)
                def _():
                    scatter_of(ci - 2, b).wait()

                build(ci, stages[b])
                pltpu.async_copy(stages[b], dacc.at[ed_all.at[ci, 1]],
                                 ssems[b], add=True)

        ci_t = _NCH - 1
        scatter_of(ci_t - 2, 0).wait()
        build(ci_t, stages[0])
        pltpu.async_copy(stages[0], dacc.at[ed_all.at[ci_t, 1]], ssems[0],
                         add=True)

        scatter_of(ci_t - 1, 1).wait()
        scatter_of(ci_t, 0).wait()
        plsc.subcore_barrier()
        _dump_acc_slice(dacc, out_hbm, c, s)

    return deg


def _make_agg(D, nact=None):
    mesh = plsc.VectorSubcoreMesh(core_axis_name="c", subcore_axis_name="s")
    nvr = D // _L
    nact = nvr if nact is None else nact

    @functools.partial(
        pl.kernel,
        mesh=mesh,
        compiler_params=pltpu.CompilerParams(use_tc_tiling_on_sc=False),
        out_type=jax.ShapeDtypeStruct((_NC, _N, D), jnp.float32),
        scratch_types=[
            pltpu.VMEM((_NCH, 2, _ECH), jnp.int32),
            pltpu.VMEM((_EPW,), jnp.float32),
            pltpu.VMEM((_ECH, D), jnp.float32),
            pltpu.VMEM((_ECH, D), jnp.float32),
            pltpu.VMEM_SHARED((_N, D), jnp.float32),
            pltpu.SemaphoreType.DMA,
            pltpu.SemaphoreType.DMA,
            pltpu.SemaphoreType.DMA,
            pltpu.SemaphoreType.DMA,
            pltpu.SemaphoreType.DMA,
            pltpu.SemaphoreType.DMA,
        ],
    )
    def agg(z_hbm, ed_hbm, ew_hbm, out_hbm, ed_all, ew_all, rows0,
            rows1, acc, gsem0, gsem1, gsem0b, gsem1b, ssem0, ssem1):
        c = lax.axis_index("c")
        s = lax.axis_index("s")
        wid = s * _NC + c
        rows = (rows0, rows1)
        gsems = (gsem0, gsem1)
        gsemsb = (gsem0b, gsem1b)
        ssems = (ssem0, ssem1)

        pltpu.sync_copy(ed_hbm.at[pl.ds(wid * _NCH, _NCH)], ed_all)
        pltpu.sync_copy(ew_hbm.at[pl.ds(wid * _EPW, _EPW)], ew_all)
        _zero_rows(rows0, _ECH, nvr)
        _zero_acc_slice(rows0, acc, s)

        _EH2 = _ECH // 2

        def gather_of(ci, b):
            return (
                pltpu.make_async_copy(
                    z_hbm.at[ed_all.at[ci, 0, pl.ds(0, _EH2)]],
                    rows[b].at[pl.ds(0, _EH2)], gsems[b]),
                pltpu.make_async_copy(
                    z_hbm.at[ed_all.at[ci, 0, pl.ds(_EH2, _EH2)]],
                    rows[b].at[pl.ds(_EH2, _EH2)], gsemsb[b]),
            )

        def gather_start(ci, b):
            for d in gather_of(ci, b):
                d.start()

        def gather_wait(ci, b):
            for d in gather_of(ci, b):
                d.wait()

        def scatter_of(ci, b):
            return pltpu.make_async_copy(rows[b], acc.at[ed_all.at[ci, 1]],
                                         ssems[b])

        gather_start(0, 0)
        plsc.subcore_barrier()

        def scale(ci, buf):
            def grp(g, _):
                ewv = ew_all[pl.ds(ci * _ECH + g * _L, _L)]
                for lane in range(_L):
                    bc = _bcast_lane(ewv, lane)
                    e = g * _L + lane
                    for j in range(nact):
                        buf[e, pl.ds(j * _L, _L)] = (
                            buf[e, pl.ds(j * _L, _L)] * bc)
                return 0
            lax.fori_loop(0, _ECH // _L, grp, 0)

        @pl.loop(0, _NCH - 1, step=2)
        def _(g):
            for b in range(2):
                ci = g + b
                nb = 1 - b

                @pl.when(ci >= 1)
                def _():
                    scatter_of(ci - 1, nb).wait()

                gather_start(ci + 1, nb)
                gather_wait(ci, b)
                scale(ci, rows[b])
                pltpu.async_copy(rows[b], acc.at[ed_all.at[ci, 1]], ssems[b],
                                 add=True)

        ci_t = _NCH - 1
        gather_wait(ci_t, 0)
        scale(ci_t, rows[0])
        pltpu.async_copy(rows[0], acc.at[ed_all.at[ci_t, 1]], ssems[0],
                         add=True)

        scatter_of(ci_t - 1, 1).wait()
        scatter_of(ci_t, 0).wait()
        plsc.subcore_barrier()
        _dump_acc_slice(acc, out_hbm, c, s)

    return agg


def _dinv_of(degp):
    deg = 1.0 + degp[0, :, 0] + degp[1, :, 0]
    return lax.rsqrt(deg)


def _mm_body(x_ref, w_ref, o_ref):
    o_ref[...] = jnp.dot(x_ref[...], w_ref[...],
                         preferred_element_type=jnp.float32)


def _scale_body(degp_ref, xw_ref, o_ref):
    dinv = _dinv_of(degp_ref[...])
    o_ref[...] = dinv[:, None] * xw_ref[...]


def _z2_body(degp_ref, aggp_ref, z1_ref, b1_ref, w2_ref, o_ref):
    dinv = _dinv_of(degp_ref[...])
    a = aggp_ref[...]
    h = dinv[:, None] * (a[0] + a[1] + z1_ref[...]) + b1_ref[...][None, :]
    h = jnp.maximum(h, 0.0)
    hw = jnp.dot(h, w2_ref[...], preferred_element_type=jnp.float32)
    o_ref[...] = jnp.concatenate(
        [dinv[:, None] * hw, jnp.zeros((_BN, _HID - _F_OUT), jnp.float32)],
        axis=1)


def _out_body(degp_ref, aggp_ref, z2_ref, b2_ref, o_ref):
    dinv = _dinv_of(degp_ref[...])
    a = aggp_ref[...]
    t = (a[0] + a[1] + z2_ref[...])[:, :_F_OUT]
    o_ref[...] = dinv[:, None] * t + b2_ref[...][None, :]


_degp_spec = pl.BlockSpec((2, _BN, _DPAD), lambda i: (0, i, 0))


def _mm(x, W):
    return pl.pallas_call(
        _mm_body,
        grid=(_GRID,),
        in_specs=[
            pl.BlockSpec((_BN, _F_IN), lambda i: (i, 0)),
            pl.BlockSpec((_F_IN, _HID), lambda i: (0, 0)),
        ],
        out_specs=pl.BlockSpec((_BN, _HID), lambda i: (i, 0)),
        out_shape=jax.ShapeDtypeStruct((_N, _HID), jnp.float32),
    )(x, W)


def _z1(degp, xw):
    return pl.pallas_call(
        _scale_body,
        grid=(_GRID,),
        in_specs=[
            _degp_spec,
            pl.BlockSpec((_BN, _HID), lambda i: (i, 0)),
        ],
        out_specs=pl.BlockSpec((_BN, _HID), lambda i: (i, 0)),
        out_shape=jax.ShapeDtypeStruct((_N, _HID), jnp.float32),
    )(degp, xw)


def _z2(degp, aggp, z1, b1, W2):
    return pl.pallas_call(
        _z2_body,
        grid=(_GRID,),
        in_specs=[
            _degp_spec,
            pl.BlockSpec((2, _BN, _HID), lambda i: (0, i, 0)),
            pl.BlockSpec((_BN, _HID), lambda i: (i, 0)),
            pl.BlockSpec((_HID,), lambda i: (0,)),
            pl.BlockSpec((_HID, _F_OUT), lambda i: (0, 0)),
        ],
        out_specs=pl.BlockSpec((_BN, _HID), lambda i: (i, 0)),
        out_shape=jax.ShapeDtypeStruct((_N, _HID), jnp.float32),
    )(degp, aggp, z1, b1, W2)


def _out(degp, aggp, z2, b2):
    return pl.pallas_call(
        _out_body,
        grid=(_GRID,),
        in_specs=[
            _degp_spec,
            pl.BlockSpec((2, _BN, _HID), lambda i: (0, i, 0)),
            pl.BlockSpec((_BN, _HID), lambda i: (i, 0)),
            pl.BlockSpec((_F_OUT,), lambda i: (0,)),
        ],
        out_specs=pl.BlockSpec((_BN, _F_OUT), lambda i: (i, 0)),
        out_shape=jax.ShapeDtypeStruct((_N, _F_OUT), jnp.float32),
    )(degp, aggp, z2, b2)


_deg_kernel = _make_deg()
_agg128 = _make_agg(_HID)
_agg64 = _make_agg(_HID, nact=_F_OUT // _L)


def kernel(x, edge_index, edge_weight, W1, b1, W2, b2):
    ed = jnp.stack([edge_index[0], edge_index[1]], axis=0)
    ed = ed.reshape(2, _NW * _NCH, _ECH).transpose(1, 0, 2)

    degp = _deg_kernel(ed, edge_weight)
    xw1 = _mm(x, W1)
    z1 = _z1(degp, xw1)
    aggp1 = _agg128(z1, ed, edge_weight)
    z2 = _z2(degp, aggp1, z1, b1, W2)
    aggp2 = _agg64(z2, ed, edge_weight)
    return _out(degp, aggp2, z2, b2)
